# speedup vs baseline: 1.0143x; 1.0143x over previous
import jax
import jax.numpy as jnp
from jax.experimental import pallas as pl
from jax.experimental.pallas import tpu as pltpu

N_NODES = 10000
D = 256
N_TX = 2500
P_MAX = 10.0

ROW_BLOCK = 1000


def _mm2_body(x_ref, w0t_ref, w1t_ref, b_ref, z0_ref, z1_ref):
    x = x_ref[...]
    z0_ref[...] = jnp.dot(x, w0t_ref[...],
                          preferred_element_type=jnp.float32) + b_ref[...]
    z1_ref[...] = jnp.dot(x, w1t_ref[...],
                          preferred_element_type=jnp.float32)


def _mm2(x, W0, W1, b):
    grid = (N_NODES // ROW_BLOCK,)
    return pl.pallas_call(
        _mm2_body,
        grid=grid,
        in_specs=[
            pl.BlockSpec((ROW_BLOCK, D), lambda i: (i, 0)),
            pl.BlockSpec((D, D), lambda i: (0, 0)),
            pl.BlockSpec((D, D), lambda i: (0, 0)),
            pl.BlockSpec((1, D), lambda i: (0, 0)),
        ],
        out_specs=[
            pl.BlockSpec((ROW_BLOCK, D), lambda i: (i, 0)),
            pl.BlockSpec((ROW_BLOCK, D), lambda i: (i, 0)),
        ],
        out_shape=[
            jax.ShapeDtypeStruct((N_NODES, D), jnp.float32),
            jax.ShapeDtypeStruct((N_NODES, D), jnp.float32),
        ],
    )(x, W0.T, W1.T, b[None, :])


def _combine_body(z0_ref, agg_ref, x_ref, y_ref):
    h = z0_ref[...] + agg_ref[...] + x_ref[...]
    y_ref[...] = jnp.where(h >= 0, h, 0.01 * h)


def _combine(z0, agg, x):
    grid = (N_NODES // ROW_BLOCK,)
    spec = pl.BlockSpec((ROW_BLOCK, D), lambda i: (i, 0))
    return pl.pallas_call(
        _combine_body,
        grid=grid,
        in_specs=[spec, spec, spec],
        out_specs=spec,
        out_shape=jax.ShapeDtypeStruct((N_NODES, D), jnp.float32),
    )(z0, agg, x)


def kernel(y, edge_index, edge_weight, transmitters_index,
           W0_0, W1_0, b_0, W0_1, W1_1, b_1, bp):
    src = edge_index[0].astype(jnp.int32)
    dst = edge_index[1].astype(jnp.int32)
    tx = transmitters_index.astype(jnp.int32)

    def layer(x, W0, W1, b):
        z0, z1 = _mm2(x, W0, W1, b)
        msg = z1[src] * edge_weight[:, None]
        agg = jax.ops.segment_sum(msg, dst, num_segments=N_NODES)
        return _combine(z0, agg, x)

    y1 = layer(y, W0_0, W1_0, b_0)
    y2 = layer(y1, W0_1, W1_1, b_1)

    s = (y2 @ bp[0])
    sums = jax.ops.segment_sum(s, tx, num_segments=N_TX)
    counts = jax.ops.segment_sum(jnp.ones((N_NODES,), jnp.float32), tx,
                                 num_segments=N_TX)
    p = P_MAX * jax.nn.sigmoid(sums / jnp.clip(counts, 1.0))
    return p[:, None]


# speedup vs baseline: 2.5100x; 2.4745x over previous
import dataclasses
import functools

import jax
import jax.numpy as jnp
from jax import lax
from jax.experimental import pallas as pl
from jax.experimental.pallas import tpu as pltpu
from jax.experimental.pallas import tpu_sc as plsc

N_NODES = 10000
N_EDGES = 160000
D = 256
HALF = 128
N_TX = 2500
TX_PAD = 2560
P_MAX = 10.0

NC = 2
NS = 16
N_PAD = 10240
ROWS_PER_TILE = N_PAD // NS
EDGES_PER_TILE = N_EDGES // NS
CHUNK = 80
N_CHUNKS = EDGES_PER_TILE // CHUNK

ROW_BLOCK = 1000


def _sc_compiler_params():
    cp = pltpu.CompilerParams()
    if "needs_layout_passes" in pltpu.CompilerParams.__dataclass_fields__:
        cp = dataclasses.replace(cp, needs_layout_passes=False)
    return cp


def _mm_body(x_ref, w0t_ref, w1t_ref, b_ref, z0_ref, z1_ref):
    x = x_ref[...]
    z0_ref[...] = jnp.dot(x, w0t_ref[...],
                          preferred_element_type=jnp.float32) + b_ref[...]
    for c in range(NC):
        z1_ref[c] = jnp.dot(x, w1t_ref[c],
                            preferred_element_type=jnp.float32)


def _mm(x, W0t, W1t_split, b):
    return pl.pallas_call(
        _mm_body,
        grid=(N_NODES // ROW_BLOCK,),
        in_specs=[
            pl.BlockSpec((ROW_BLOCK, D), lambda i: (i, 0)),
            pl.BlockSpec((D, D), lambda i: (0, 0)),
            pl.BlockSpec((NC, D, HALF), lambda i: (0, 0, 0)),
            pl.BlockSpec((1, D), lambda i: (0, 0)),
        ],
        out_specs=[
            pl.BlockSpec((ROW_BLOCK, D), lambda i: (i, 0)),
            pl.BlockSpec((NC, ROW_BLOCK, HALF), lambda i: (0, i, 0)),
        ],
        out_shape=[
            jax.ShapeDtypeStruct((N_NODES, D), jnp.float32),
            jax.ShapeDtypeStruct((NC, N_NODES, HALF), jnp.float32),
        ],
    )(x, W0t, W1t_split, b[None, :])


def _edge_agg_body(z1_hbm, src_hbm, dst_hbm, w_hbm, zeros_hbm, out_hbm,
                   sidx_v, didx_v, w_v, rows_v, accum, gsem):
    c = lax.axis_index("c")
    s = lax.axis_index("s")

    pltpu.sync_copy(zeros_hbm.at[pl.ds(s * ROWS_PER_TILE, ROWS_PER_TILE)],
                    accum.at[pl.ds(s * ROWS_PER_TILE, ROWS_PER_TILE)])
    plsc.subcore_barrier()

    @pl.loop(0, N_CHUNKS)
    def _(k):
        base = s * EDGES_PER_TILE + k * CHUNK
        pltpu.sync_copy(src_hbm.at[pl.ds(base, CHUNK)], sidx_v)
        pltpu.sync_copy(w_hbm.at[pl.ds(base, CHUNK)], w_v)
        pltpu.async_copy(z1_hbm.at[c].at[sidx_v], rows_v, gsem).wait()

        @pl.loop(0, CHUNK)
        def _(e):
            widx = lax.broadcast(e, (16,))
            wv = plsc.load_gather(w_v, [widx])
            for j in range(HALF // 16):
                sl = rows_v.at[e, pl.ds(j * 16, 16)]
                sl[...] = sl[...] * wv

        pltpu.sync_copy(dst_hbm.at[pl.ds(base, CHUNK)], didx_v)
        pltpu.sync_copy(rows_v, accum.at[didx_v], add=True)

    plsc.subcore_barrier()
    pltpu.sync_copy(accum.at[pl.ds(s * ROWS_PER_TILE, ROWS_PER_TILE)],
                    out_hbm.at[c, pl.ds(s * ROWS_PER_TILE, ROWS_PER_TILE)])


def _edge_agg(z1_split, src, dst, w, zeros_pad):
    mesh = plsc.VectorSubcoreMesh(core_axis_name="c", subcore_axis_name="s")
    f = pl.kernel(
        _edge_agg_body,
        out_type=jax.ShapeDtypeStruct((NC, N_PAD, HALF), jnp.float32),
        mesh=mesh,
        scratch_types=[
            pltpu.VMEM((CHUNK,), jnp.int32),
            pltpu.VMEM((CHUNK,), jnp.int32),
            pltpu.VMEM((CHUNK,), jnp.float32),
            pltpu.VMEM((CHUNK, HALF), jnp.float32),
            pltpu.VMEM_SHARED((N_PAD, HALF), jnp.float32),
            pltpu.SemaphoreType.DMA,
        ],
        compiler_params=_sc_compiler_params(),
    )
    return f(z1_split, src, dst, w, zeros_pad)


def _combine_body(z0_ref, a0_ref, a1_ref, x_ref, y_ref):
    h0 = z0_ref[:, :HALF] + a0_ref[0] + x_ref[:, :HALF]
    h1 = z0_ref[:, HALF:] + a1_ref[0] + x_ref[:, HALF:]
    y_ref[:, :HALF] = jnp.where(h0 >= 0, h0, 0.01 * h0)
    y_ref[:, HALF:] = jnp.where(h1 >= 0, h1, 0.01 * h1)


def _combine(z0, agg, x):
    spec = pl.BlockSpec((ROW_BLOCK, D), lambda i: (i, 0))
    return pl.pallas_call(
        _combine_body,
        grid=(N_NODES // ROW_BLOCK,),
        in_specs=[
            spec,
            pl.BlockSpec((1, ROW_BLOCK, HALF), lambda i: (0, i, 0)),
            pl.BlockSpec((1, ROW_BLOCK, HALF), lambda i: (1, i, 0)),
            spec,
        ],
        out_specs=spec,
        out_shape=jax.ShapeDtypeStruct((N_NODES, D), jnp.float32),
    )(z0, agg, agg, x)


def _combine_readout_body(z0_ref, a0_ref, a1_ref, x_ref, bp_ref, s_ref):
    h0 = z0_ref[:, :HALF] + a0_ref[0] + x_ref[:, :HALF]
    h1 = z0_ref[:, HALF:] + a1_ref[0] + x_ref[:, HALF:]
    y0 = jnp.where(h0 >= 0, h0, 0.01 * h0)
    y1 = jnp.where(h1 >= 0, h1, 0.01 * h1)
    sv = jnp.sum(y0 * bp_ref[:, :HALF], axis=1) + \
        jnp.sum(y1 * bp_ref[:, HALF:], axis=1)
    s_ref[...] = sv[:, None]


def _combine_readout(z0, agg, x, bp):
    spec = pl.BlockSpec((ROW_BLOCK, D), lambda i: (i, 0))
    s2d = pl.pallas_call(
        _combine_readout_body,
        grid=(N_NODES // ROW_BLOCK,),
        in_specs=[
            spec,
            pl.BlockSpec((1, ROW_BLOCK, HALF), lambda i: (0, i, 0)),
            pl.BlockSpec((1, ROW_BLOCK, HALF), lambda i: (1, i, 0)),
            spec,
            pl.BlockSpec((1, D), lambda i: (0, 0)),
        ],
        out_specs=pl.BlockSpec((ROW_BLOCK, 1), lambda i: (i, 0)),
        out_shape=jax.ShapeDtypeStruct((N_NODES, 1), jnp.float32),
    )(z0, agg, agg, x, bp)
    return s2d.reshape(N_NODES)


NODE_CHUNKS = N_NODES // CHUNK
MAX_CHUNKS_PER_W = (NODE_CHUNKS + NC * NS - 1) // (NC * NS)


def _readout_body(s_hbm, tx_hbm, out_hbm, sv_v, txv_v, hist, cnt, ones_v):
    c = lax.axis_index("c")
    s = lax.axis_index("s")
    wid = s * NC + c

    ones_v[...] = jnp.full((16,), 1.0, jnp.float32)

    @pl.loop(0, TX_PAD // 16)
    def _(i):
        z = jnp.zeros((16,), jnp.float32)
        hist.at[pl.ds(i * 16, 16)][...] = z
        cnt.at[pl.ds(i * 16, 16)][...] = z

    @pl.loop(0, MAX_CHUNKS_PER_W)
    def _(kk):
        k = kk * (NC * NS) + wid

        @pl.when(k < NODE_CHUNKS)
        def _():
            base = k * CHUNK
            pltpu.sync_copy(s_hbm.at[pl.ds(base, CHUNK)], sv_v)
            pltpu.sync_copy(tx_hbm.at[pl.ds(base, CHUNK)], txv_v)

            @pl.loop(0, CHUNK // 16)
            def _(g):
                iv = txv_v[pl.ds(g * 16, 16)]
                vv = sv_v[pl.ds(g * 16, 16)]
                plsc.addupdate_scatter(hist, [iv], vv)
                plsc.addupdate_scatter(cnt, [iv], ones_v[...])

    pltpu.sync_copy(hist, out_hbm.at[wid, 0])
    pltpu.sync_copy(cnt, out_hbm.at[wid, 1])


def _readout(s_vals, tx):
    mesh = plsc.VectorSubcoreMesh(core_axis_name="c", subcore_axis_name="s")
    f = pl.kernel(
        _readout_body,
        out_type=jax.ShapeDtypeStruct((NC * NS, 2, TX_PAD), jnp.float32),
        mesh=mesh,
        scratch_types=[
            pltpu.VMEM((CHUNK,), jnp.float32),
            pltpu.VMEM((CHUNK,), jnp.int32),
            pltpu.VMEM((TX_PAD,), jnp.float32),
            pltpu.VMEM((TX_PAD,), jnp.float32),
            pltpu.VMEM((16,), jnp.float32),
        ],
        compiler_params=_sc_compiler_params(),
    )
    return f(s_vals, tx)


def _finalize_body(h_ref, p_ref):
    sums = jnp.sum(h_ref[:, 0, :], axis=0)
    counts = jnp.sum(h_ref[:, 1, :], axis=0)
    m = sums / jnp.maximum(counts, 1.0)
    p_ref[...] = (P_MAX * jax.nn.sigmoid(m))[None, :]


def _finalize(hists):
    return pl.pallas_call(
        _finalize_body,
        out_shape=jax.ShapeDtypeStruct((1, TX_PAD), jnp.float32),
    )(hists)


def kernel(y, edge_index, edge_weight, transmitters_index,
           W0_0, W1_0, b_0, W0_1, W1_1, b_1, bp):
    src = edge_index[0].astype(jnp.int32)
    dst = edge_index[1].astype(jnp.int32)
    tx = transmitters_index.astype(jnp.int32)
    w = edge_weight.astype(jnp.float32)
    zeros_pad = jnp.zeros((N_PAD, HALF), jnp.float32)

    def prep(W1):
        return W1.T.reshape(D, NC, HALF).transpose(1, 0, 2)

    z0, z1 = _mm(y, W0_0.T, prep(W1_0), b_0)
    agg = _edge_agg(z1, src, dst, w, zeros_pad)
    y1 = _combine(z0, agg, y)

    z0b, z1b = _mm(y1, W0_1.T, prep(W1_1), b_1)
    aggb = _edge_agg(z1b, src, dst, w, zeros_pad)
    s_vals = _combine_readout(z0b, aggb, y1, bp)

    hists = _readout(s_vals, tx)
    p = _finalize(hists)
    return p[0, :N_TX][:, None]


# speedup vs baseline: 2.8599x; 1.1394x over previous
import dataclasses
import functools

import jax
import jax.numpy as jnp
from jax import lax
from jax.experimental import pallas as pl
from jax.experimental.pallas import tpu as pltpu
from jax.experimental.pallas import tpu_sc as plsc

N_NODES = 10000
N_EDGES = 160000
D = 256
HALF = 128
N_TX = 2500
TX_PAD = 2560
P_MAX = 10.0

NC = 2
NS = 16
N_PAD = 10240
ROWS_PER_TILE = N_PAD // NS
EDGES_PER_TILE = N_EDGES // NS
CHUNK = 80
N_CHUNKS = EDGES_PER_TILE // CHUNK

ROW_BLOCK = 1000


def _sc_compiler_params():
    cp = pltpu.CompilerParams()
    if "needs_layout_passes" in pltpu.CompilerParams.__dataclass_fields__:
        cp = dataclasses.replace(cp, needs_layout_passes=False)
    return cp


def _mm_body(x_ref, w0t_ref, w1t_ref, b_ref, z0_ref, z1_ref):
    x = x_ref[...]
    z0_ref[...] = jnp.dot(x, w0t_ref[...],
                          preferred_element_type=jnp.float32) + b_ref[...]
    for c in range(NC):
        z1_ref[c] = jnp.dot(x, w1t_ref[c],
                            preferred_element_type=jnp.float32)


def _mm(x, W0t, W1t_split, b):
    return pl.pallas_call(
        _mm_body,
        grid=(N_NODES // ROW_BLOCK,),
        in_specs=[
            pl.BlockSpec((ROW_BLOCK, D), lambda i: (i, 0)),
            pl.BlockSpec((D, D), lambda i: (0, 0)),
            pl.BlockSpec((NC, D, HALF), lambda i: (0, 0, 0)),
            pl.BlockSpec((1, D), lambda i: (0, 0)),
        ],
        out_specs=[
            pl.BlockSpec((ROW_BLOCK, D), lambda i: (i, 0)),
            pl.BlockSpec((NC, ROW_BLOCK, HALF), lambda i: (0, i, 0)),
        ],
        out_shape=[
            jax.ShapeDtypeStruct((N_NODES, D), jnp.float32),
            jax.ShapeDtypeStruct((NC, N_NODES, HALF), jnp.float32),
        ],
    )(x, W0t, W1t_split, b[None, :])


def _edge_agg_body(z1_hbm, src_hbm, dst_hbm, w_hbm, zeros_hbm, out_hbm,
                   sidx, didx, wbuf, rows_g, rows_s, accum,
                   gsem0, gsem1, ssem0, ssem1, isem0, isem1):
    c = lax.axis_index("c")
    s = lax.axis_index("s")
    gsem = (gsem0, gsem1)
    ssem = (ssem0, ssem1)
    isem = (isem0, isem1)

    pltpu.sync_copy(zeros_hbm.at[pl.ds(s * ROWS_PER_TILE, ROWS_PER_TILE)],
                    accum.at[pl.ds(s * ROWS_PER_TILE, ROWS_PER_TILE)])
    plsc.subcore_barrier()

    def chunk_base(k):
        return s * EDGES_PER_TILE + k * CHUNK

    def multiply(b2, b4):
        @pl.loop(0, CHUNK)
        def _(e):
            widx = lax.broadcast(e, (16,))
            wv = plsc.load_gather(wbuf.at[b4], [widx])
            for j in range(HALF // 16):
                rows_s.at[b2, e, pl.ds(j * 16, 16)][...] = \
                    rows_g.at[b2, e, pl.ds(j * 16, 16)][...] * wv

    for b in range(2):
        base = chunk_base(b)
        pltpu.sync_copy(src_hbm.at[pl.ds(base, CHUNK)], sidx.at[b])
        pltpu.sync_copy(dst_hbm.at[pl.ds(base, CHUNK)], didx.at[b])
        pltpu.sync_copy(w_hbm.at[pl.ds(base, CHUNK)], wbuf.at[b])
        pltpu.async_copy(z1_hbm.at[c].at[sidx.at[b]], rows_g.at[b], gsem[b])

    @pl.loop(0, (N_CHUNKS + 3) // 4)
    def _(k):
        for b in range(4):
            b2 = b % 2
            nb4 = (b + 2) % 4
            cur = k * 4 + b

            @pl.when(cur < N_CHUNKS)
            def _():
                pltpu.make_async_copy(z1_hbm.at[c].at[sidx.at[b]],
                                      rows_g.at[b2], gsem[b2]).wait()
                if b >= 2:
                    pltpu.make_async_copy(
                        rows_s.at[b2], accum.at[didx.at[b - 2]],
                        ssem[b2]).wait()
                else:
                    @pl.when(k > 0)
                    def _():
                        pltpu.make_async_copy(
                            rows_s.at[b2], accum.at[didx.at[b + 2]],
                            ssem[b2]).wait()

                @pl.when(cur + 2 < N_CHUNKS)
                def _():
                    nbase = chunk_base(cur + 2)
                    pltpu.async_copy(src_hbm.at[pl.ds(nbase, CHUNK)],
                                     sidx.at[nb4], isem[b2])
                    pltpu.async_copy(dst_hbm.at[pl.ds(nbase, CHUNK)],
                                     didx.at[nb4], isem[b2])
                    pltpu.async_copy(w_hbm.at[pl.ds(nbase, CHUNK)],
                                     wbuf.at[nb4], isem[b2])

                multiply(b2, b)

                pltpu.async_copy(rows_s.at[b2], accum.at[didx.at[b]],
                                 ssem[b2], add=True)

                @pl.when(cur + 2 < N_CHUNKS)
                def _():
                    nbase = chunk_base(cur + 2)
                    pltpu.make_async_copy(src_hbm.at[pl.ds(nbase, CHUNK)],
                                          sidx.at[nb4], isem[b2]).wait()
                    pltpu.make_async_copy(dst_hbm.at[pl.ds(nbase, CHUNK)],
                                          didx.at[nb4], isem[b2]).wait()
                    pltpu.make_async_copy(w_hbm.at[pl.ds(nbase, CHUNK)],
                                          wbuf.at[nb4], isem[b2]).wait()
                    pltpu.async_copy(z1_hbm.at[c].at[sidx.at[nb4]],
                                     rows_g.at[b2], gsem[b2])

    pltpu.make_async_copy(rows_s.at[1], accum.at[didx.at[3]], ssem[1]).wait()
    pltpu.make_async_copy(rows_s.at[0], accum.at[didx.at[0]], ssem[0]).wait()

    plsc.subcore_barrier()
    pltpu.sync_copy(accum.at[pl.ds(s * ROWS_PER_TILE, ROWS_PER_TILE)],
                    out_hbm.at[c, pl.ds(s * ROWS_PER_TILE, ROWS_PER_TILE)])


def _edge_agg(z1_split, src, dst, w, zeros_pad):
    mesh = plsc.VectorSubcoreMesh(core_axis_name="c", subcore_axis_name="s")
    f = pl.kernel(
        _edge_agg_body,
        out_type=jax.ShapeDtypeStruct((NC, N_PAD, HALF), jnp.float32),
        mesh=mesh,
        scratch_types=[
            pltpu.VMEM((4, CHUNK), jnp.int32),
            pltpu.VMEM((4, CHUNK), jnp.int32),
            pltpu.VMEM((4, CHUNK), jnp.float32),
            pltpu.VMEM((2, CHUNK, HALF), jnp.float32),
            pltpu.VMEM((2, CHUNK, HALF), jnp.float32),
            pltpu.VMEM_SHARED((N_PAD, HALF), jnp.float32),
            pltpu.SemaphoreType.DMA,
            pltpu.SemaphoreType.DMA,
            pltpu.SemaphoreType.DMA,
            pltpu.SemaphoreType.DMA,
            pltpu.SemaphoreType.DMA,
            pltpu.SemaphoreType.DMA,
        ],
        compiler_params=_sc_compiler_params(),
    )
    return f(z1_split, src, dst, w, zeros_pad)


def _combine_body(z0_ref, a0_ref, a1_ref, x_ref, y_ref):
    h0 = z0_ref[:, :HALF] + a0_ref[0] + x_ref[:, :HALF]
    h1 = z0_ref[:, HALF:] + a1_ref[0] + x_ref[:, HALF:]
    y_ref[:, :HALF] = jnp.where(h0 >= 0, h0, 0.01 * h0)
    y_ref[:, HALF:] = jnp.where(h1 >= 0, h1, 0.01 * h1)


def _combine(z0, agg, x):
    spec = pl.BlockSpec((ROW_BLOCK, D), lambda i: (i, 0))
    return pl.pallas_call(
        _combine_body,
        grid=(N_NODES // ROW_BLOCK,),
        in_specs=[
            spec,
            pl.BlockSpec((1, ROW_BLOCK, HALF), lambda i: (0, i, 0)),
            pl.BlockSpec((1, ROW_BLOCK, HALF), lambda i: (1, i, 0)),
            spec,
        ],
        out_specs=spec,
        out_shape=jax.ShapeDtypeStruct((N_NODES, D), jnp.float32),
    )(z0, agg, agg, x)


def _combine_readout_body(z0_ref, a0_ref, a1_ref, x_ref, bp_ref, s_ref):
    h0 = z0_ref[:, :HALF] + a0_ref[0] + x_ref[:, :HALF]
    h1 = z0_ref[:, HALF:] + a1_ref[0] + x_ref[:, HALF:]
    y0 = jnp.where(h0 >= 0, h0, 0.01 * h0)
    y1 = jnp.where(h1 >= 0, h1, 0.01 * h1)
    sv = jnp.sum(y0 * bp_ref[:, :HALF], axis=1) + \
        jnp.sum(y1 * bp_ref[:, HALF:], axis=1)
    s_ref[...] = sv[:, None]


def _combine_readout(z0, agg, x, bp):
    spec = pl.BlockSpec((ROW_BLOCK, D), lambda i: (i, 0))
    s2d = pl.pallas_call(
        _combine_readout_body,
        grid=(N_NODES // ROW_BLOCK,),
        in_specs=[
            spec,
            pl.BlockSpec((1, ROW_BLOCK, HALF), lambda i: (0, i, 0)),
            pl.BlockSpec((1, ROW_BLOCK, HALF), lambda i: (1, i, 0)),
            spec,
            pl.BlockSpec((1, D), lambda i: (0, 0)),
        ],
        out_specs=pl.BlockSpec((ROW_BLOCK, 1), lambda i: (i, 0)),
        out_shape=jax.ShapeDtypeStruct((N_NODES, 1), jnp.float32),
    )(z0, agg, agg, x, bp)
    return s2d.reshape(N_NODES)


NODE_CHUNKS = N_NODES // CHUNK
MAX_CHUNKS_PER_W = (NODE_CHUNKS + NC * NS - 1) // (NC * NS)


def _readout_body(s_hbm, tx_hbm, out_hbm, sv_v, txv_v, hist, cnt, ones_v):
    c = lax.axis_index("c")
    s = lax.axis_index("s")
    wid = s * NC + c

    ones_v[...] = jnp.full((16,), 1.0, jnp.float32)

    @pl.loop(0, TX_PAD // 16)
    def _(i):
        z = jnp.zeros((16,), jnp.float32)
        hist.at[pl.ds(i * 16, 16)][...] = z
        cnt.at[pl.ds(i * 16, 16)][...] = z

    @pl.loop(0, MAX_CHUNKS_PER_W)
    def _(kk):
        k = kk * (NC * NS) + wid

        @pl.when(k < NODE_CHUNKS)
        def _():
            base = k * CHUNK
            pltpu.sync_copy(s_hbm.at[pl.ds(base, CHUNK)], sv_v)
            pltpu.sync_copy(tx_hbm.at[pl.ds(base, CHUNK)], txv_v)

            @pl.loop(0, CHUNK // 16)
            def _(g):
                iv = txv_v[pl.ds(g * 16, 16)]
                vv = sv_v[pl.ds(g * 16, 16)]
                plsc.addupdate_scatter(hist, [iv], vv)
                plsc.addupdate_scatter(cnt, [iv], ones_v[...])

    pltpu.sync_copy(hist, out_hbm.at[wid, 0])
    pltpu.sync_copy(cnt, out_hbm.at[wid, 1])


def _readout(s_vals, tx):
    mesh = plsc.VectorSubcoreMesh(core_axis_name="c", subcore_axis_name="s")
    f = pl.kernel(
        _readout_body,
        out_type=jax.ShapeDtypeStruct((NC * NS, 2, TX_PAD), jnp.float32),
        mesh=mesh,
        scratch_types=[
            pltpu.VMEM((CHUNK,), jnp.float32),
            pltpu.VMEM((CHUNK,), jnp.int32),
            pltpu.VMEM((TX_PAD,), jnp.float32),
            pltpu.VMEM((TX_PAD,), jnp.float32),
            pltpu.VMEM((16,), jnp.float32),
        ],
        compiler_params=_sc_compiler_params(),
    )
    return f(s_vals, tx)


def _finalize_body(h_ref, p_ref):
    sums = jnp.sum(h_ref[:, 0, :], axis=0)
    counts = jnp.sum(h_ref[:, 1, :], axis=0)
    m = sums / jnp.maximum(counts, 1.0)
    p_ref[...] = (P_MAX * jax.nn.sigmoid(m))[None, :]


def _finalize(hists):
    return pl.pallas_call(
        _finalize_body,
        out_shape=jax.ShapeDtypeStruct((1, TX_PAD), jnp.float32),
    )(hists)


def kernel(y, edge_index, edge_weight, transmitters_index,
           W0_0, W1_0, b_0, W0_1, W1_1, b_1, bp):
    src = edge_index[0].astype(jnp.int32)
    dst = edge_index[1].astype(jnp.int32)
    tx = transmitters_index.astype(jnp.int32)
    w = edge_weight.astype(jnp.float32)
    zeros_pad = jnp.zeros((N_PAD, HALF), jnp.float32)

    def prep(W1):
        return W1.T.reshape(D, NC, HALF).transpose(1, 0, 2)

    z0, z1 = _mm(y, W0_0.T, prep(W1_0), b_0)
    agg = _edge_agg(z1, src, dst, w, zeros_pad)
    y1 = _combine(z0, agg, y)

    z0b, z1b = _mm(y1, W0_1.T, prep(W1_1), b_1)
    aggb = _edge_agg(z1b, src, dst, w, zeros_pad)
    s_vals = _combine_readout(z0b, aggb, y1, bp)

    hists = _readout(s_vals, tx)
    p = _finalize(hists)
    return p[0, :N_TX][:, None]


# speedup vs baseline: 6.6446x; 2.3234x over previous
import dataclasses
import functools

import jax
import jax.numpy as jnp
import numpy as np
from jax import lax
from jax.experimental import pallas as pl
from jax.experimental.pallas import tpu as pltpu
from jax.experimental.pallas import tpu_sc as plsc

N_NODES = 10000
N_EDGES = 160000
D = 256
HALF = 128
N_TX = 2500
TX_PAD = 2560
P_MAX = 10.0

NC = 2
NS = 16
N_PAD = 10240
ROWS_PER_TILE = N_PAD // NS
EDGES_PER_TILE = N_EDGES // NS
CHUNK = 80
N_CHUNKS = EDGES_PER_TILE // CHUNK

ROW_BLOCK = 1000


def _sc_compiler_params():
    cp = pltpu.CompilerParams()
    if "needs_layout_passes" in pltpu.CompilerParams.__dataclass_fields__:
        cp = dataclasses.replace(cp, needs_layout_passes=False)
    return cp


def _mm_body(x_ref, w0t_ref, w1t_ref, b_ref, z0_ref, z1_ref):
    x = x_ref[...]
    z0_ref[...] = jnp.dot(x, w0t_ref[...],
                          preferred_element_type=jnp.float32) + b_ref[...]
    for c in range(NC):
        z1_ref[c] = jnp.dot(x, w1t_ref[c],
                            preferred_element_type=jnp.float32)


def _mm(x, W0t, W1t_split, b):
    return pl.pallas_call(
        _mm_body,
        grid=(N_NODES // ROW_BLOCK,),
        in_specs=[
            pl.BlockSpec((ROW_BLOCK, D), lambda i: (i, 0)),
            pl.BlockSpec((D, D), lambda i: (0, 0)),
            pl.BlockSpec((NC, D, HALF), lambda i: (0, 0, 0)),
            pl.BlockSpec((1, D), lambda i: (0, 0)),
        ],
        out_specs=[
            pl.BlockSpec((ROW_BLOCK, D), lambda i: (i, 0)),
            pl.BlockSpec((NC, ROW_BLOCK, HALF), lambda i: (0, i, 0)),
        ],
        out_shape=[
            jax.ShapeDtypeStruct((N_NODES, D), jnp.float32),
            jax.ShapeDtypeStruct((NC, N_NODES, HALF), jnp.float32),
        ],
    )(x, W0t, W1t_split, b[None, :])


def _edge_agg_body(z1_hbm, src_hbm, dst_hbm, w_hbm, zeros_hbm, out_hbm,
                   sidx, didx, wbuf, rows_g, rows_s, accum,
                   gsem0, gsem1, ssem0, ssem1, isem0, isem1):
    c = lax.axis_index("c")
    s = lax.axis_index("s")
    gsem = (gsem0, gsem1)
    ssem = (ssem0, ssem1)
    isem = (isem0, isem1)

    pltpu.sync_copy(zeros_hbm.at[pl.ds(s * ROWS_PER_TILE, ROWS_PER_TILE)],
                    accum.at[pl.ds(s * ROWS_PER_TILE, ROWS_PER_TILE)])
    plsc.subcore_barrier()

    def chunk_base(k):
        return s * EDGES_PER_TILE + k * CHUNK

    def multiply(b2, b4):
        @pl.loop(0, CHUNK // 16)
        def _(g):
            for e in range(16):
                row = g * 16 + e
                widx = lax.broadcast(row, (16,))
                wv = plsc.load_gather(wbuf.at[b4], [widx])
                for j in range(HALF // 16):
                    rows_s.at[b2, row, pl.ds(j * 16, 16)][...] = \
                        rows_g.at[b2, row, pl.ds(j * 16, 16)][...] * wv

    for b in range(2):
        base = chunk_base(b)
        pltpu.sync_copy(src_hbm.at[pl.ds(base, CHUNK)], sidx.at[b])
        pltpu.sync_copy(dst_hbm.at[pl.ds(base, CHUNK)], didx.at[b])
        pltpu.sync_copy(w_hbm.at[pl.ds(base, CHUNK)], wbuf.at[b])
        pltpu.async_copy(z1_hbm.at[c].at[sidx.at[b]], rows_g.at[b], gsem[b])

    @pl.loop(0, (N_CHUNKS + 3) // 4)
    def _(k):
        for b in range(4):
            b2 = b % 2
            nb4 = (b + 2) % 4
            cur = k * 4 + b

            @pl.when(cur < N_CHUNKS)
            def _():
                pltpu.make_async_copy(z1_hbm.at[c].at[sidx.at[b]],
                                      rows_g.at[b2], gsem[b2]).wait()
                if b >= 2:
                    pltpu.make_async_copy(
                        rows_s.at[b2], accum.at[didx.at[b - 2]],
                        ssem[b2]).wait()
                else:
                    @pl.when(k > 0)
                    def _():
                        pltpu.make_async_copy(
                            rows_s.at[b2], accum.at[didx.at[b + 2]],
                            ssem[b2]).wait()

                @pl.when(cur + 2 < N_CHUNKS)
                def _():
                    nbase = chunk_base(cur + 2)
                    pltpu.async_copy(src_hbm.at[pl.ds(nbase, CHUNK)],
                                     sidx.at[nb4], isem[b2])
                    pltpu.async_copy(dst_hbm.at[pl.ds(nbase, CHUNK)],
                                     didx.at[nb4], isem[b2])
                    pltpu.async_copy(w_hbm.at[pl.ds(nbase, CHUNK)],
                                     wbuf.at[nb4], isem[b2])

                multiply(b2, b)

                pltpu.async_copy(rows_s.at[b2], accum.at[didx.at[b]],
                                 ssem[b2], add=True)

                @pl.when(cur + 2 < N_CHUNKS)
                def _():
                    nbase = chunk_base(cur + 2)
                    pltpu.make_async_copy(src_hbm.at[pl.ds(nbase, CHUNK)],
                                          sidx.at[nb4], isem[b2]).wait()
                    pltpu.make_async_copy(dst_hbm.at[pl.ds(nbase, CHUNK)],
                                          didx.at[nb4], isem[b2]).wait()
                    pltpu.make_async_copy(w_hbm.at[pl.ds(nbase, CHUNK)],
                                          wbuf.at[nb4], isem[b2]).wait()
                    pltpu.async_copy(z1_hbm.at[c].at[sidx.at[nb4]],
                                     rows_g.at[b2], gsem[b2])

    pltpu.make_async_copy(rows_s.at[1], accum.at[didx.at[3]], ssem[1]).wait()
    pltpu.make_async_copy(rows_s.at[0], accum.at[didx.at[0]], ssem[0]).wait()

    plsc.subcore_barrier()
    pltpu.sync_copy(accum.at[pl.ds(s * ROWS_PER_TILE, ROWS_PER_TILE)],
                    out_hbm.at[c, pl.ds(s * ROWS_PER_TILE, ROWS_PER_TILE)])


def _edge_agg(z1_split, src, dst, w, zeros_pad):
    mesh = plsc.VectorSubcoreMesh(core_axis_name="c", subcore_axis_name="s")
    f = pl.kernel(
        _edge_agg_body,
        out_type=jax.ShapeDtypeStruct((NC, N_PAD, HALF), jnp.float32),
        mesh=mesh,
        scratch_types=[
            pltpu.VMEM((4, CHUNK), jnp.int32),
            pltpu.VMEM((4, CHUNK), jnp.int32),
            pltpu.VMEM((4, CHUNK), jnp.float32),
            pltpu.VMEM((2, CHUNK, HALF), jnp.float32),
            pltpu.VMEM((2, CHUNK, HALF), jnp.float32),
            pltpu.VMEM_SHARED((N_PAD, HALF), jnp.float32),
            pltpu.SemaphoreType.DMA,
            pltpu.SemaphoreType.DMA,
            pltpu.SemaphoreType.DMA,
            pltpu.SemaphoreType.DMA,
            pltpu.SemaphoreType.DMA,
            pltpu.SemaphoreType.DMA,
        ],
        compiler_params=_sc_compiler_params(),
    )
    return f(z1_split, src, dst, w, zeros_pad)


def _combine_body(z0_ref, a0_ref, a1_ref, x_ref, y_ref):
    h0 = z0_ref[:, :HALF] + a0_ref[0] + x_ref[:, :HALF]
    h1 = z0_ref[:, HALF:] + a1_ref[0] + x_ref[:, HALF:]
    y_ref[:, :HALF] = jnp.where(h0 >= 0, h0, 0.01 * h0)
    y_ref[:, HALF:] = jnp.where(h1 >= 0, h1, 0.01 * h1)


def _combine(z0, agg, x):
    spec = pl.BlockSpec((ROW_BLOCK, D), lambda i: (i, 0))
    return pl.pallas_call(
        _combine_body,
        grid=(N_NODES // ROW_BLOCK,),
        in_specs=[
            spec,
            pl.BlockSpec((1, ROW_BLOCK, HALF), lambda i: (0, i, 0)),
            pl.BlockSpec((1, ROW_BLOCK, HALF), lambda i: (1, i, 0)),
            spec,
        ],
        out_specs=spec,
        out_shape=jax.ShapeDtypeStruct((N_NODES, D), jnp.float32),
    )(z0, agg, agg, x)


def _combine_readout_body(z0_ref, a0_ref, a1_ref, x_ref, bp_ref, s_ref):
    h0 = z0_ref[:, :HALF] + a0_ref[0] + x_ref[:, :HALF]
    h1 = z0_ref[:, HALF:] + a1_ref[0] + x_ref[:, HALF:]
    y0 = jnp.where(h0 >= 0, h0, 0.01 * h0)
    y1 = jnp.where(h1 >= 0, h1, 0.01 * h1)
    sv = jnp.sum(y0 * bp_ref[:, :HALF], axis=1) + \
        jnp.sum(y1 * bp_ref[:, HALF:], axis=1)
    s_ref[...] = sv[:, None]


def _combine_readout(z0, agg, x, bp):
    spec = pl.BlockSpec((ROW_BLOCK, D), lambda i: (i, 0))
    s2d = pl.pallas_call(
        _combine_readout_body,
        grid=(N_NODES // ROW_BLOCK,),
        in_specs=[
            spec,
            pl.BlockSpec((1, ROW_BLOCK, HALF), lambda i: (0, i, 0)),
            pl.BlockSpec((1, ROW_BLOCK, HALF), lambda i: (1, i, 0)),
            spec,
            pl.BlockSpec((1, D), lambda i: (0, 0)),
        ],
        out_specs=pl.BlockSpec((ROW_BLOCK, 1), lambda i: (i, 0)),
        out_shape=jax.ShapeDtypeStruct((N_NODES, 1), jnp.float32),
    )(z0, agg, agg, x, bp)
    return s2d.reshape(N_NODES)


NODE_CHUNKS = N_NODES // CHUNK
MAX_CHUNKS_PER_W = (NODE_CHUNKS + NC * NS - 1) // (NC * NS)


def _readout_body(s_hbm, tx_hbm, out_hbm, sv_v, txv_v, hist, cnt, ones_v):
    c = lax.axis_index("c")
    s = lax.axis_index("s")
    wid = s * NC + c

    ones_v[...] = jnp.full((16,), 1.0, jnp.float32)

    @pl.loop(0, TX_PAD // 16)
    def _(i):
        z = jnp.zeros((16,), jnp.float32)
        hist.at[pl.ds(i * 16, 16)][...] = z
        cnt.at[pl.ds(i * 16, 16)][...] = z

    @pl.loop(0, MAX_CHUNKS_PER_W)
    def _(kk):
        k = kk * (NC * NS) + wid

        @pl.when(k < NODE_CHUNKS)
        def _():
            base = k * CHUNK
            pltpu.sync_copy(s_hbm.at[pl.ds(base, CHUNK)], sv_v)
            pltpu.sync_copy(tx_hbm.at[pl.ds(base, CHUNK)], txv_v)

            @pl.loop(0, CHUNK // 16)
            def _(g):
                iv = txv_v[pl.ds(g * 16, 16)]
                vv = sv_v[pl.ds(g * 16, 16)]
                plsc.addupdate_scatter(hist, [iv], vv)
                plsc.addupdate_scatter(cnt, [iv], ones_v[...])

    pltpu.sync_copy(hist, out_hbm.at[wid, 0])
    pltpu.sync_copy(cnt, out_hbm.at[wid, 1])


def _readout(s_vals, tx):
    mesh = plsc.VectorSubcoreMesh(core_axis_name="c", subcore_axis_name="s")
    f = pl.kernel(
        _readout_body,
        out_type=jax.ShapeDtypeStruct((NC * NS, 2, TX_PAD), jnp.float32),
        mesh=mesh,
        scratch_types=[
            pltpu.VMEM((CHUNK,), jnp.float32),
            pltpu.VMEM((CHUNK,), jnp.int32),
            pltpu.VMEM((TX_PAD,), jnp.float32),
            pltpu.VMEM((TX_PAD,), jnp.float32),
            pltpu.VMEM((16,), jnp.float32),
        ],
        compiler_params=_sc_compiler_params(),
    )
    return f(s_vals, tx)


def _finalize_body(h_ref, p_ref):
    sums = jnp.sum(h_ref[:, 0, :], axis=0)
    counts = jnp.sum(h_ref[:, 1, :], axis=0)
    m = sums / jnp.maximum(counts, 1.0)
    p_ref[...] = (P_MAX * jax.nn.sigmoid(m))[None, :]


def _finalize(hists):
    return pl.pallas_call(
        _finalize_body,
        out_shape=jax.ShapeDtypeStruct((1, TX_PAD), jnp.float32),
    )(hists)


def kernel(y, edge_index, edge_weight, transmitters_index,
           W0_0, W1_0, b_0, W0_1, W1_1, b_1, bp):
    src = edge_index[0].astype(jnp.int32)
    dst = edge_index[1].astype(jnp.int32)
    tx = transmitters_index.astype(jnp.int32)
    w = edge_weight.astype(jnp.float32)
    zeros_pad = jnp.zeros((N_PAD, HALF), jnp.float32)

    def prep(W1):
        return W1.T.reshape(D, NC, HALF).transpose(1, 0, 2)

    z0, z1 = _mm(y, W0_0.T, prep(W1_0), b_0)
    agg = _edge_agg(z1, src, dst, w, zeros_pad)
    y1 = _combine(z0, agg, y)

    z0b, z1b = _mm(y1, W0_1.T, prep(W1_1), b_1)
    aggb = _edge_agg(z1b, src, dst, w, zeros_pad)
    s_vals = _combine_readout(z0b, aggb, y1, bp)

    hists = _readout(s_vals, tx)
    p = _finalize(hists)
    return p[0, :N_TX][:, None]


# speedup vs baseline: 6.8154x; 1.0257x over previous
import dataclasses
import functools

import jax
import jax.numpy as jnp
import numpy as np
from jax import lax
from jax.experimental import pallas as pl
from jax.experimental.pallas import tpu as pltpu
from jax.experimental.pallas import tpu_sc as plsc

N_NODES = 10000
N_EDGES = 160000
D = 256
HALF = 128
N_TX = 2500
TX_PAD = 2560
P_MAX = 10.0

NC = 2
NS = 16
N_PAD = 10240
ROWS_PER_TILE = N_PAD // NS
EDGES_PER_TILE = N_EDGES // NS
CHUNK = 80
N_CHUNKS = EDGES_PER_TILE // CHUNK

ROW_BLOCK = 1000


def _sc_compiler_params():
    cp = pltpu.CompilerParams()
    if "needs_layout_passes" in pltpu.CompilerParams.__dataclass_fields__:
        cp = dataclasses.replace(cp, needs_layout_passes=False)
    return cp


def _mm_z1_body(x_ref, w1t_ref, z1_ref):
    x = x_ref[...]
    for c in range(NC):
        z1_ref[c] = jnp.dot(x, w1t_ref[c],
                            preferred_element_type=jnp.float32)


def _mm_z1(x, W1t_split):
    return pl.pallas_call(
        _mm_z1_body,
        grid=(N_NODES // ROW_BLOCK,),
        in_specs=[
            pl.BlockSpec((ROW_BLOCK, D), lambda i: (i, 0)),
            pl.BlockSpec((NC, D, HALF), lambda i: (0, 0, 0)),
        ],
        out_specs=pl.BlockSpec((NC, ROW_BLOCK, HALF), lambda i: (0, i, 0)),
        out_shape=jax.ShapeDtypeStruct((NC, N_NODES, HALF), jnp.float32),
    )(x, W1t_split)


def _mm_z0_body(x_ref, w0t_ref, b_ref, z0_ref):
    z0_ref[...] = jnp.dot(x_ref[...], w0t_ref[...],
                          preferred_element_type=jnp.float32) + b_ref[...]


def _mm_z0(x, W0t, b):
    return pl.pallas_call(
        _mm_z0_body,
        grid=(N_NODES // ROW_BLOCK,),
        in_specs=[
            pl.BlockSpec((ROW_BLOCK, D), lambda i: (i, 0)),
            pl.BlockSpec((D, D), lambda i: (0, 0)),
            pl.BlockSpec((1, D), lambda i: (0, 0)),
        ],
        out_specs=pl.BlockSpec((ROW_BLOCK, D), lambda i: (i, 0)),
        out_shape=jax.ShapeDtypeStruct((N_NODES, D), jnp.float32),
    )(x, W0t, b[None, :])


def _combine_mm_body(z0_ref, a0_ref, a1_ref, x_ref, w0t_ref, w1t_ref, b_ref,
                     y_ref, z0b_ref, z1b_ref):
    h0 = z0_ref[:, :HALF] + a0_ref[0] + x_ref[:, :HALF]
    h1 = z0_ref[:, HALF:] + a1_ref[0] + x_ref[:, HALF:]
    y0 = jnp.where(h0 >= 0, h0, 0.01 * h0)
    y1 = jnp.where(h1 >= 0, h1, 0.01 * h1)
    y = jnp.concatenate([y0, y1], axis=1)
    y_ref[...] = y
    z0b_ref[...] = jnp.dot(y, w0t_ref[...],
                           preferred_element_type=jnp.float32) + b_ref[...]
    for c in range(NC):
        z1b_ref[c] = jnp.dot(y, w1t_ref[c],
                             preferred_element_type=jnp.float32)


def _combine_mm(z0, agg, x, W0t, W1t_split, b):
    spec = pl.BlockSpec((ROW_BLOCK, D), lambda i: (i, 0))
    return pl.pallas_call(
        _combine_mm_body,
        grid=(N_NODES // ROW_BLOCK,),
        in_specs=[
            spec,
            pl.BlockSpec((1, ROW_BLOCK, HALF), lambda i: (0, i, 0)),
            pl.BlockSpec((1, ROW_BLOCK, HALF), lambda i: (1, i, 0)),
            spec,
            pl.BlockSpec((D, D), lambda i: (0, 0)),
            pl.BlockSpec((NC, D, HALF), lambda i: (0, 0, 0)),
            pl.BlockSpec((1, D), lambda i: (0, 0)),
        ],
        out_specs=[
            spec,
            spec,
            pl.BlockSpec((NC, ROW_BLOCK, HALF), lambda i: (0, i, 0)),
        ],
        out_shape=[
            jax.ShapeDtypeStruct((N_NODES, D), jnp.float32),
            jax.ShapeDtypeStruct((N_NODES, D), jnp.float32),
            jax.ShapeDtypeStruct((NC, N_NODES, HALF), jnp.float32),
        ],
    )(z0, agg, agg, x, W0t, W1t_split, b[None, :])


def _edge_agg_body(z1_hbm, src_hbm, dst_hbm, w_hbm, zeros_hbm, out_hbm,
                   sidx, didx, wbuf, rows_g, rows_s, accum,
                   gsem0, gsem1, ssem0, ssem1, isem0, isem1):
    c = lax.axis_index("c")
    s = lax.axis_index("s")
    gsem = (gsem0, gsem1)
    ssem = (ssem0, ssem1)
    isem = (isem0, isem1)

    pltpu.sync_copy(zeros_hbm.at[pl.ds(s * ROWS_PER_TILE, ROWS_PER_TILE)],
                    accum.at[pl.ds(s * ROWS_PER_TILE, ROWS_PER_TILE)])
    plsc.subcore_barrier()

    def chunk_base(k):
        return s * EDGES_PER_TILE + k * CHUNK

    def multiply(b2, b4):
        @pl.loop(0, CHUNK // 16)
        def _(g):
            for e in range(16):
                row = g * 16 + e
                widx = lax.broadcast(row, (16,))
                wv = plsc.load_gather(wbuf.at[b4], [widx])
                for j in range(HALF // 16):
                    rows_s.at[b2, row, pl.ds(j * 16, 16)][...] = \
                        rows_g.at[b2, row, pl.ds(j * 16, 16)][...] * wv

    for b in range(2):
        base = chunk_base(b)
        pltpu.sync_copy(src_hbm.at[pl.ds(base, CHUNK)], sidx.at[b])
        pltpu.sync_copy(dst_hbm.at[pl.ds(base, CHUNK)], didx.at[b])
        pltpu.sync_copy(w_hbm.at[pl.ds(base, CHUNK)], wbuf.at[b])
        pltpu.async_copy(z1_hbm.at[c].at[sidx.at[b]], rows_g.at[b], gsem[b])

    @pl.loop(0, (N_CHUNKS + 3) // 4)
    def _(k):
        for b in range(4):
            b2 = b % 2
            nb4 = (b + 2) % 4
            cur = k * 4 + b

            @pl.when(cur < N_CHUNKS)
            def _():
                pltpu.make_async_copy(z1_hbm.at[c].at[sidx.at[b]],
                                      rows_g.at[b2], gsem[b2]).wait()
                if b >= 2:
                    pltpu.make_async_copy(
                        rows_s.at[b2], accum.at[didx.at[b - 2]],
                        ssem[b2]).wait()
                else:
                    @pl.when(k > 0)
                    def _():
                        pltpu.make_async_copy(
                            rows_s.at[b2], accum.at[didx.at[b + 2]],
                            ssem[b2]).wait()

                @pl.when(cur + 2 < N_CHUNKS)
                def _():
                    nbase = chunk_base(cur + 2)
                    pltpu.async_copy(src_hbm.at[pl.ds(nbase, CHUNK)],
                                     sidx.at[nb4], isem[b2])
                    pltpu.async_copy(dst_hbm.at[pl.ds(nbase, CHUNK)],
                                     didx.at[nb4], isem[b2])
                    pltpu.async_copy(w_hbm.at[pl.ds(nbase, CHUNK)],
                                     wbuf.at[nb4], isem[b2])

                multiply(b2, b)

                pltpu.async_copy(rows_s.at[b2], accum.at[didx.at[b]],
                                 ssem[b2], add=True)

                @pl.when(cur + 2 < N_CHUNKS)
                def _():
                    nbase = chunk_base(cur + 2)
                    pltpu.make_async_copy(src_hbm.at[pl.ds(nbase, CHUNK)],
                                          sidx.at[nb4], isem[b2]).wait()
                    pltpu.make_async_copy(dst_hbm.at[pl.ds(nbase, CHUNK)],
                                          didx.at[nb4], isem[b2]).wait()
                    pltpu.make_async_copy(w_hbm.at[pl.ds(nbase, CHUNK)],
                                          wbuf.at[nb4], isem[b2]).wait()
                    pltpu.async_copy(z1_hbm.at[c].at[sidx.at[nb4]],
                                     rows_g.at[b2], gsem[b2])

    pltpu.make_async_copy(rows_s.at[1], accum.at[didx.at[3]], ssem[1]).wait()
    pltpu.make_async_copy(rows_s.at[0], accum.at[didx.at[0]], ssem[0]).wait()

    plsc.subcore_barrier()
    pltpu.sync_copy(accum.at[pl.ds(s * ROWS_PER_TILE, ROWS_PER_TILE)],
                    out_hbm.at[c, pl.ds(s * ROWS_PER_TILE, ROWS_PER_TILE)])


def _edge_agg(z1_split, src, dst, w, zeros_pad):
    mesh = plsc.VectorSubcoreMesh(core_axis_name="c", subcore_axis_name="s")
    f = pl.kernel(
        _edge_agg_body,
        out_type=jax.ShapeDtypeStruct((NC, N_PAD, HALF), jnp.float32),
        mesh=mesh,
        scratch_types=[
            pltpu.VMEM((4, CHUNK), jnp.int32),
            pltpu.VMEM((4, CHUNK), jnp.int32),
            pltpu.VMEM((4, CHUNK), jnp.float32),
            pltpu.VMEM((2, CHUNK, HALF), jnp.float32),
            pltpu.VMEM((2, CHUNK, HALF), jnp.float32),
            pltpu.VMEM_SHARED((N_PAD, HALF), jnp.float32),
            pltpu.SemaphoreType.DMA,
            pltpu.SemaphoreType.DMA,
            pltpu.SemaphoreType.DMA,
            pltpu.SemaphoreType.DMA,
            pltpu.SemaphoreType.DMA,
            pltpu.SemaphoreType.DMA,
        ],
        compiler_params=_sc_compiler_params(),
    )
    return f(z1_split, src, dst, w, zeros_pad)


def _combine_readout_body(z0_ref, a0_ref, a1_ref, x_ref, bp_ref, s_ref):
    h0 = z0_ref[:, :HALF] + a0_ref[0] + x_ref[:, :HALF]
    h1 = z0_ref[:, HALF:] + a1_ref[0] + x_ref[:, HALF:]
    y0 = jnp.where(h0 >= 0, h0, 0.01 * h0)
    y1 = jnp.where(h1 >= 0, h1, 0.01 * h1)
    sv = jnp.sum(y0 * bp_ref[:, :HALF], axis=1) + \
        jnp.sum(y1 * bp_ref[:, HALF:], axis=1)
    s_ref[...] = sv[:, None]


def _combine_readout(z0, agg, x, bp):
    spec = pl.BlockSpec((ROW_BLOCK, D), lambda i: (i, 0))
    s2d = pl.pallas_call(
        _combine_readout_body,
        grid=(N_NODES // ROW_BLOCK,),
        in_specs=[
            spec,
            pl.BlockSpec((1, ROW_BLOCK, HALF), lambda i: (0, i, 0)),
            pl.BlockSpec((1, ROW_BLOCK, HALF), lambda i: (1, i, 0)),
            spec,
            pl.BlockSpec((1, D), lambda i: (0, 0)),
        ],
        out_specs=pl.BlockSpec((ROW_BLOCK, 1), lambda i: (i, 0)),
        out_shape=jax.ShapeDtypeStruct((N_NODES, 1), jnp.float32),
    )(z0, agg, agg, x, bp)
    return s2d.reshape(N_NODES)


NODE_CHUNKS = N_NODES // CHUNK
MAX_CHUNKS_PER_W = (NODE_CHUNKS + NC * NS - 1) // (NC * NS)


def _readout_body(s_hbm, tx_hbm, out_hbm, sv_v, txv_v, hist, cnt, ones_v):
    c = lax.axis_index("c")
    s = lax.axis_index("s")
    wid = s * NC + c

    ones_v[...] = jnp.full((16,), 1.0, jnp.float32)

    @pl.loop(0, TX_PAD // 16)
    def _(i):
        z = jnp.zeros((16,), jnp.float32)
        hist.at[pl.ds(i * 16, 16)][...] = z
        cnt.at[pl.ds(i * 16, 16)][...] = z

    @pl.loop(0, MAX_CHUNKS_PER_W)
    def _(kk):
        k = kk * (NC * NS) + wid

        @pl.when(k < NODE_CHUNKS)
        def _():
            base = k * CHUNK
            pltpu.sync_copy(s_hbm.at[pl.ds(base, CHUNK)], sv_v)
            pltpu.sync_copy(tx_hbm.at[pl.ds(base, CHUNK)], txv_v)

            @pl.loop(0, CHUNK // 16)
            def _(g):
                iv = txv_v[pl.ds(g * 16, 16)]
                vv = sv_v[pl.ds(g * 16, 16)]
                plsc.addupdate_scatter(hist, [iv], vv)
                plsc.addupdate_scatter(cnt, [iv], ones_v[...])

    pltpu.sync_copy(hist, out_hbm.at[wid, 0])
    pltpu.sync_copy(cnt, out_hbm.at[wid, 1])


def _readout(s_vals, tx):
    mesh = plsc.VectorSubcoreMesh(core_axis_name="c", subcore_axis_name="s")
    f = pl.kernel(
        _readout_body,
        out_type=jax.ShapeDtypeStruct((NC * NS, 2, TX_PAD), jnp.float32),
        mesh=mesh,
        scratch_types=[
            pltpu.VMEM((CHUNK,), jnp.float32),
            pltpu.VMEM((CHUNK,), jnp.int32),
            pltpu.VMEM((TX_PAD,), jnp.float32),
            pltpu.VMEM((TX_PAD,), jnp.float32),
            pltpu.VMEM((16,), jnp.float32),
        ],
        compiler_params=_sc_compiler_params(),
    )
    return f(s_vals, tx)


def _finalize_body(h_ref, p_ref):
    sums = jnp.sum(h_ref[:, 0, :], axis=0)
    counts = jnp.sum(h_ref[:, 1, :], axis=0)
    m = sums / jnp.maximum(counts, 1.0)
    p_ref[...] = (P_MAX * jax.nn.sigmoid(m))[None, :]


def _finalize(hists):
    return pl.pallas_call(
        _finalize_body,
        out_shape=jax.ShapeDtypeStruct((1, TX_PAD), jnp.float32),
    )(hists)


def kernel(y, edge_index, edge_weight, transmitters_index,
           W0_0, W1_0, b_0, W0_1, W1_1, b_1, bp):
    src = edge_index[0].astype(jnp.int32)
    dst = edge_index[1].astype(jnp.int32)
    tx = transmitters_index.astype(jnp.int32)
    w = edge_weight.astype(jnp.float32)
    zeros_pad = jnp.zeros((N_PAD, HALF), jnp.float32)

    def prep(W1):
        return W1.T.reshape(D, NC, HALF).transpose(1, 0, 2)

    z1 = _mm_z1(y, prep(W1_0))
    agg = _edge_agg(z1, src, dst, w, zeros_pad)
    z0 = _mm_z0(y, W0_0.T, b_0)

    y1, z0b, z1b = _combine_mm(z0, agg, y, W0_1.T, prep(W1_1), b_1)
    aggb = _edge_agg(z1b, src, dst, w, zeros_pad)
    s_vals = _combine_readout(z0b, aggb, y1, bp)

    hists = _readout(s_vals, tx)
    p = _finalize(hists)
    return p[0, :N_TX][:, None]


# speedup vs baseline: 6.9409x; 1.0184x over previous
import dataclasses
import functools

import jax
import jax.numpy as jnp
import numpy as np
from jax import lax
from jax.experimental import pallas as pl
from jax.experimental.pallas import tpu as pltpu
from jax.experimental.pallas import tpu_sc as plsc

N_NODES = 10000
N_EDGES = 160000
D = 256
HALF = 128
N_TX = 2500
TX_PAD = 2560
P_MAX = 10.0

NC = 2
NS = 16
N_PAD = 10240
ROWS_PER_TILE = N_PAD // NS
CHUNK = 80
TOTAL_CHUNKS = N_EDGES // CHUNK
MAX_T = TOTAL_CHUNKS // NS

ROW_BLOCK = 1000


def _sc_compiler_params():
    cp = pltpu.CompilerParams()
    if "needs_layout_passes" in pltpu.CompilerParams.__dataclass_fields__:
        cp = dataclasses.replace(cp, needs_layout_passes=False)
    return cp


def _mm_z1_body(x_ref, w1t_ref, z1_ref):
    x = x_ref[...]
    for c in range(NC):
        z1_ref[c] = jnp.dot(x, w1t_ref[c],
                            preferred_element_type=jnp.float32)


def _mm_z1(x, W1t_split):
    return pl.pallas_call(
        _mm_z1_body,
        grid=(N_NODES // ROW_BLOCK,),
        in_specs=[
            pl.BlockSpec((ROW_BLOCK, D), lambda i: (i, 0)),
            pl.BlockSpec((NC, D, HALF), lambda i: (0, 0, 0)),
        ],
        out_specs=pl.BlockSpec((NC, ROW_BLOCK, HALF), lambda i: (0, i, 0)),
        out_shape=jax.ShapeDtypeStruct((NC, N_NODES, HALF), jnp.float32),
    )(x, W1t_split)


def _mm_z0_body(x_ref, w0t_ref, b_ref, z0_ref):
    z0_ref[...] = jnp.dot(x_ref[...], w0t_ref[...],
                          preferred_element_type=jnp.float32) + b_ref[...]


def _mm_z0(x, W0t, b):
    return pl.pallas_call(
        _mm_z0_body,
        grid=(N_NODES // ROW_BLOCK,),
        in_specs=[
            pl.BlockSpec((ROW_BLOCK, D), lambda i: (i, 0)),
            pl.BlockSpec((D, D), lambda i: (0, 0)),
            pl.BlockSpec((1, D), lambda i: (0, 0)),
        ],
        out_specs=pl.BlockSpec((ROW_BLOCK, D), lambda i: (i, 0)),
        out_shape=jax.ShapeDtypeStruct((N_NODES, D), jnp.float32),
    )(x, W0t, b[None, :])


def _combine_mm_body(z0_ref, a0_ref, a1_ref, x_ref, w0t_ref, w1t_ref, b_ref,
                     y_ref, z0b_ref, z1b_ref):
    h0 = z0_ref[:, :HALF] + a0_ref[0] + x_ref[:, :HALF]
    h1 = z0_ref[:, HALF:] + a1_ref[0] + x_ref[:, HALF:]
    y0 = jnp.where(h0 >= 0, h0, 0.01 * h0)
    y1 = jnp.where(h1 >= 0, h1, 0.01 * h1)
    y = jnp.concatenate([y0, y1], axis=1)
    y_ref[...] = y
    z0b_ref[...] = jnp.dot(y, w0t_ref[...],
                           preferred_element_type=jnp.float32) + b_ref[...]
    for c in range(NC):
        z1b_ref[c] = jnp.dot(y, w1t_ref[c],
                             preferred_element_type=jnp.float32)


def _combine_mm(z0, agg, x, W0t, W1t_split, b):
    spec = pl.BlockSpec((ROW_BLOCK, D), lambda i: (i, 0))
    return pl.pallas_call(
        _combine_mm_body,
        grid=(N_NODES // ROW_BLOCK,),
        in_specs=[
            spec,
            pl.BlockSpec((1, ROW_BLOCK, HALF), lambda i: (0, i, 0)),
            pl.BlockSpec((1, ROW_BLOCK, HALF), lambda i: (1, i, 0)),
            spec,
            pl.BlockSpec((D, D), lambda i: (0, 0)),
            pl.BlockSpec((NC, D, HALF), lambda i: (0, 0, 0)),
            pl.BlockSpec((1, D), lambda i: (0, 0)),
        ],
        out_specs=[
            spec,
            spec,
            pl.BlockSpec((NC, ROW_BLOCK, HALF), lambda i: (0, i, 0)),
        ],
        out_shape=[
            jax.ShapeDtypeStruct((N_NODES, D), jnp.float32),
            jax.ShapeDtypeStruct((N_NODES, D), jnp.float32),
            jax.ShapeDtypeStruct((NC, N_NODES, HALF), jnp.float32),
        ],
    )(z0, agg, agg, x, W0t, W1t_split, b[None, :])


def _edge_agg_body(z1_hbm, src_hbm, dst_hbm, w_hbm, zeros_hbm, out_hbm,
                   sidx, didx, wbuf, rows, accum,
                   gsem0, gsem1, gsem2, gsem3,
                   ssem0, ssem1, ssem2, ssem3, isem0, isem1, zsem):
    c = lax.axis_index("c")
    s = lax.axis_index("s")
    gsem = (gsem0, gsem1, gsem2, gsem3)
    ssem = (ssem0, ssem1, ssem2, ssem3)
    isem = (isem0, isem1)

    pltpu.async_copy(zeros_hbm.at[pl.ds(s * ROWS_PER_TILE, ROWS_PER_TILE)],
                     accum.at[pl.ds(s * ROWS_PER_TILE, ROWS_PER_TILE)], zsem)

    def base_of(t):
        return (s + t * NS) * CHUNK

    def valid(t):
        return s + t * NS < TOTAL_CHUNKS

    def multiply(slot):
        @pl.loop(0, CHUNK // 16)
        def _(g):
            for e in range(16):
                row = g * 16 + e
                widx = lax.broadcast(row, (16,))
                wv = plsc.load_gather(wbuf.at[slot], [widx])
                for j in range(HALF // 16):
                    rows.at[slot, row, pl.ds(j * 16, 16)][...] = \
                        rows.at[slot, row, pl.ds(j * 16, 16)][...] * wv

    for b in range(2):
        base = base_of(b)
        pltpu.sync_copy(src_hbm.at[pl.ds(base, CHUNK)], sidx.at[b])
        pltpu.sync_copy(dst_hbm.at[pl.ds(base, CHUNK)], didx.at[b])
        pltpu.sync_copy(w_hbm.at[pl.ds(base, CHUNK)], wbuf.at[b])
        pltpu.async_copy(z1_hbm.at[c].at[sidx.at[b]], rows.at[b], gsem[b])

    pltpu.make_async_copy(
        zeros_hbm.at[pl.ds(s * ROWS_PER_TILE, ROWS_PER_TILE)],
        accum.at[pl.ds(s * ROWS_PER_TILE, ROWS_PER_TILE)], zsem).wait()
    plsc.subcore_barrier()

    @pl.loop(0, (MAX_T + 3) // 4)
    def _(k):
        for b in range(4):
            pf = (b + 2) % 4
            t = k * 4 + b

            @pl.when(valid(t))
            def _():
                if b >= 2:
                    pltpu.make_async_copy(
                        rows.at[pf], accum.at[didx.at[pf]], ssem[pf]).wait()
                else:
                    @pl.when(k > 0)
                    def _():
                        pltpu.make_async_copy(
                            rows.at[pf], accum.at[didx.at[pf]],
                            ssem[pf]).wait()

                @pl.when(valid(t + 2))
                def _():
                    nbase = base_of(t + 2)
                    pltpu.async_copy(src_hbm.at[pl.ds(nbase, CHUNK)],
                                     sidx.at[pf], isem[b % 2])
                    pltpu.async_copy(dst_hbm.at[pl.ds(nbase, CHUNK)],
                                     didx.at[pf], isem[b % 2])
                    pltpu.async_copy(w_hbm.at[pl.ds(nbase, CHUNK)],
                                     wbuf.at[pf], isem[b % 2])

                pltpu.make_async_copy(z1_hbm.at[c].at[sidx.at[b]],
                                      rows.at[b], gsem[b]).wait()
                multiply(b)
                pltpu.async_copy(rows.at[b], accum.at[didx.at[b]],
                                 ssem[b], add=True)

                @pl.when(valid(t + 2))
                def _():
                    nbase = base_of(t + 2)
                    pltpu.make_async_copy(src_hbm.at[pl.ds(nbase, CHUNK)],
                                          sidx.at[pf], isem[b % 2]).wait()
                    pltpu.make_async_copy(dst_hbm.at[pl.ds(nbase, CHUNK)],
                                          didx.at[pf], isem[b % 2]).wait()
                    pltpu.make_async_copy(w_hbm.at[pl.ds(nbase, CHUNK)],
                                          wbuf.at[pf], isem[b % 2]).wait()
                    pltpu.async_copy(z1_hbm.at[c].at[sidx.at[pf]],
                                     rows.at[pf], gsem[pf])

    pltpu.make_async_copy(rows.at[3], accum.at[didx.at[3]], ssem[3]).wait()
    pltpu.make_async_copy(rows.at[0], accum.at[didx.at[0]], ssem[0]).wait()

    plsc.subcore_barrier()
    pltpu.sync_copy(accum.at[pl.ds(s * ROWS_PER_TILE, ROWS_PER_TILE)],
                    out_hbm.at[c, pl.ds(s * ROWS_PER_TILE, ROWS_PER_TILE)])


def _edge_agg(z1_split, src, dst, w, zeros_pad):
    mesh = plsc.VectorSubcoreMesh(core_axis_name="c", subcore_axis_name="s")
    f = pl.kernel(
        _edge_agg_body,
        out_type=jax.ShapeDtypeStruct((NC, N_PAD, HALF), jnp.float32),
        mesh=mesh,
        scratch_types=[
            pltpu.VMEM((4, CHUNK), jnp.int32),
            pltpu.VMEM((4, CHUNK), jnp.int32),
            pltpu.VMEM((4, CHUNK), jnp.float32),
            pltpu.VMEM((4, CHUNK, HALF), jnp.float32),
            pltpu.VMEM_SHARED((N_PAD, HALF), jnp.float32),
            pltpu.SemaphoreType.DMA,
            pltpu.SemaphoreType.DMA,
            pltpu.SemaphoreType.DMA,
            pltpu.SemaphoreType.DMA,
            pltpu.SemaphoreType.DMA,
            pltpu.SemaphoreType.DMA,
            pltpu.SemaphoreType.DMA,
            pltpu.SemaphoreType.DMA,
            pltpu.SemaphoreType.DMA,
            pltpu.SemaphoreType.DMA,
            pltpu.SemaphoreType.DMA,
        ],
        compiler_params=_sc_compiler_params(),
    )
    return f(z1_split, src, dst, w, zeros_pad)


def _combine_readout_body(z0_ref, a0_ref, a1_ref, x_ref, bp_ref, s_ref):
    h0 = z0_ref[:, :HALF] + a0_ref[0] + x_ref[:, :HALF]
    h1 = z0_ref[:, HALF:] + a1_ref[0] + x_ref[:, HALF:]
    y0 = jnp.where(h0 >= 0, h0, 0.01 * h0)
    y1 = jnp.where(h1 >= 0, h1, 0.01 * h1)
    sv = jnp.sum(y0 * bp_ref[:, :HALF], axis=1) + \
        jnp.sum(y1 * bp_ref[:, HALF:], axis=1)
    s_ref[...] = sv[:, None]


def _combine_readout(z0, agg, x, bp):
    spec = pl.BlockSpec((ROW_BLOCK, D), lambda i: (i, 0))
    s2d = pl.pallas_call(
        _combine_readout_body,
        grid=(N_NODES // ROW_BLOCK,),
        in_specs=[
            spec,
            pl.BlockSpec((1, ROW_BLOCK, HALF), lambda i: (0, i, 0)),
            pl.BlockSpec((1, ROW_BLOCK, HALF), lambda i: (1, i, 0)),
            spec,
            pl.BlockSpec((1, D), lambda i: (0, 0)),
        ],
        out_specs=pl.BlockSpec((ROW_BLOCK, 1), lambda i: (i, 0)),
        out_shape=jax.ShapeDtypeStruct((N_NODES, 1), jnp.float32),
    )(z0, agg, agg, x, bp)
    return s2d.reshape(N_NODES)


NODE_CHUNKS = N_NODES // CHUNK
MAX_CHUNKS_PER_W = (NODE_CHUNKS + NC * NS - 1) // (NC * NS)


def _readout_body(s_hbm, tx_hbm, out_hbm, sv_v, txv_v, hist, cnt, ones_v):
    c = lax.axis_index("c")
    s = lax.axis_index("s")
    wid = s * NC + c

    ones_v[...] = jnp.full((16,), 1.0, jnp.float32)

    @pl.loop(0, TX_PAD // 16)
    def _(i):
        z = jnp.zeros((16,), jnp.float32)
        hist.at[pl.ds(i * 16, 16)][...] = z
        cnt.at[pl.ds(i * 16, 16)][...] = z

    @pl.loop(0, MAX_CHUNKS_PER_W)
    def _(kk):
        k = kk * (NC * NS) + wid

        @pl.when(k < NODE_CHUNKS)
        def _():
            base = k * CHUNK
            pltpu.sync_copy(s_hbm.at[pl.ds(base, CHUNK)], sv_v)
            pltpu.sync_copy(tx_hbm.at[pl.ds(base, CHUNK)], txv_v)

            @pl.loop(0, CHUNK // 16)
            def _(g):
                iv = txv_v[pl.ds(g * 16, 16)]
                vv = sv_v[pl.ds(g * 16, 16)]
                plsc.addupdate_scatter(hist, [iv], vv)
                plsc.addupdate_scatter(cnt, [iv], ones_v[...])

    pltpu.sync_copy(hist, out_hbm.at[wid, 0])
    pltpu.sync_copy(cnt, out_hbm.at[wid, 1])


def _readout(s_vals, tx):
    mesh = plsc.VectorSubcoreMesh(core_axis_name="c", subcore_axis_name="s")
    f = pl.kernel(
        _readout_body,
        out_type=jax.ShapeDtypeStruct((NC * NS, 2, TX_PAD), jnp.float32),
        mesh=mesh,
        scratch_types=[
            pltpu.VMEM((CHUNK,), jnp.float32),
            pltpu.VMEM((CHUNK,), jnp.int32),
            pltpu.VMEM((TX_PAD,), jnp.float32),
            pltpu.VMEM((TX_PAD,), jnp.float32),
            pltpu.VMEM((16,), jnp.float32),
        ],
        compiler_params=_sc_compiler_params(),
    )
    return f(s_vals, tx)


def _finalize_body(h_ref, p_ref):
    sums = jnp.sum(h_ref[:, 0, :], axis=0)
    counts = jnp.sum(h_ref[:, 1, :], axis=0)
    m = sums / jnp.maximum(counts, 1.0)
    p_ref[...] = (P_MAX * jax.nn.sigmoid(m))[None, :]


def _finalize(hists):
    return pl.pallas_call(
        _finalize_body,
        out_shape=jax.ShapeDtypeStruct((1, TX_PAD), jnp.float32),
    )(hists)


def kernel(y, edge_index, edge_weight, transmitters_index,
           W0_0, W1_0, b_0, W0_1, W1_1, b_1, bp):
    src = edge_index[0].astype(jnp.int32)
    dst = edge_index[1].astype(jnp.int32)
    tx = transmitters_index.astype(jnp.int32)
    w = edge_weight.astype(jnp.float32)
    zeros_pad = jnp.zeros((N_PAD, HALF), jnp.float32)

    def prep(W1):
        return W1.T.reshape(D, NC, HALF).transpose(1, 0, 2)

    z1 = _mm_z1(y, prep(W1_0))
    agg = _edge_agg(z1, src, dst, w, zeros_pad)
    z0 = _mm_z0(y, W0_0.T, b_0)

    y1, z0b, z1b = _combine_mm(z0, agg, y, W0_1.T, prep(W1_1), b_1)
    aggb = _edge_agg(z1b, src, dst, w, zeros_pad)
    s_vals = _combine_readout(z0b, aggb, y1, bp)

    hists = _readout(s_vals, tx)
    p = _finalize(hists)
    return p[0, :N_TX][:, None]
